# Initial kernel scaffold; baseline (speedup 1.0000x reference)
import jax
import jax.numpy as jnp
from jax.experimental import pallas as pl


def kernel(verts, edges, segment_ids, w0_1, b0_1, w1_1, b1_1, w0_2, b0_2, w1_2, b1_2, w0_3, b0_3, w1_3, b1_3, fc1_w, fc1_b, fc2_w, fc2_b):
    raise NotImplementedError("write your pallas kernel here")


# speedup vs baseline: 2.1970x; 2.1970x over previous
import functools

import jax
import jax.numpy as jnp
from jax import lax
from jax.experimental import pallas as pl
from jax.experimental.pallas import tpu as pltpu
from jax.experimental.pallas import tpu_sc as plsc

N = 10000
E = 160000
H = 256
HH = 128
NB = 10
NC = 10
R = 1000
GRID = N // R

NTILES = 16
NCORES = 2
CH = 128
NCH = 160
E2 = 2 * E
E2P = NTILES * NCH * CH
ACC_ROWS = 10240
PAD_ROW = N + 64
ZROWS = 64
OUT_PER_TILE = N // NTILES


def _gs_body(y1s_hbm, src2_hbm, dst_hbm, out_hbm, acc, srcv, dstv, rows, zbuf, sem):
    c = lax.axis_index("c")
    s = lax.axis_index("s")

    def _zb(i, _):
        r = i // (HH // 16)
        l = i % (HH // 16)
        zbuf[r, pl.ds(l * 16, 16)] = jnp.zeros((16,), jnp.float32)
        return _
    lax.fori_loop(0, ZROWS * (HH // 16), _zb, 0)

    rows_per_tile = ACC_ROWS // NTILES

    def _zero(i, _):
        pltpu.sync_copy(zbuf, acc.at[pl.ds(s * rows_per_tile + i * ZROWS, ZROWS)])
        return _
    lax.fori_loop(0, rows_per_tile // ZROWS, _zero, 0)

    plsc.subcore_barrier()

    def _chunk(k, _):
        base = (s * NCH + k) * CH
        pltpu.sync_copy(src2_hbm.at[pl.ds(c * E2P + base, CH)], srcv)
        pltpu.sync_copy(dst_hbm.at[pl.ds(base, CH)], dstv)
        pltpu.async_copy(y1s_hbm.at[srcv], rows, sem).wait()
        pltpu.sync_copy(rows, acc.at[dstv], add=True)
        return _
    lax.fori_loop(0, NCH, _chunk, 0)

    plsc.subcore_barrier()

    @pl.when(s < NTILES - 1)
    def _():
        pltpu.sync_copy(acc.at[pl.ds(s * 640, 640)],
                        out_hbm.at[pl.ds(c * N + s * 640, 640)])

    @pl.when(s == NTILES - 1)
    def _():
        pltpu.sync_copy(acc.at[pl.ds((NTILES - 1) * 640, 400)],
                        out_hbm.at[pl.ds(c * N + (NTILES - 1) * 640, 400)])


_gs_call = functools.partial(
    pl.kernel,
    out_type=jax.ShapeDtypeStruct((NCORES * N, HH), jnp.float32),
    mesh=plsc.VectorSubcoreMesh(core_axis_name="c", subcore_axis_name="s"),
    scratch_types=[
        pltpu.VMEM_SHARED((ACC_ROWS, HH), jnp.float32),
        pltpu.VMEM((CH,), jnp.int32),
        pltpu.VMEM((CH,), jnp.int32),
        pltpu.VMEM((CH, HH), jnp.float32),
        pltpu.VMEM((ZROWS, HH), jnp.float32),
        pltpu.SemaphoreType.DMA,
    ],
)(_gs_body)


def _layer1_body(x_ref, w_ref, b_ref, y0_ref, y1s_ref):
    y = jnp.dot(x_ref[...], w_ref[...], preferred_element_type=jnp.float32)
    y = y + b_ref[...]
    y0_ref[...] = y[:, :H]
    y1s_ref[0] = y[:, H:H + HH]
    y1s_ref[1] = y[:, H + HH:]


def _layerB_body(y0p_ref, gsp_ref, w_ref, b_ref, y0_ref, y1s_ref):
    gs = jnp.concatenate([gsp_ref[0], gsp_ref[1]], axis=1)
    x = jnp.maximum(y0p_ref[...] + gs, 0.0)
    y = jnp.dot(x, w_ref[...], preferred_element_type=jnp.float32)
    y = y + b_ref[...]
    y0_ref[...] = y[:, :H]
    y1s_ref[0] = y[:, H:H + HH]
    y1s_ref[1] = y[:, H + HH:]


def _layer1(x, w, b):
    return pl.pallas_call(
        _layer1_body,
        grid=(GRID,),
        in_specs=[
            pl.BlockSpec((R, x.shape[1]), lambda i: (i, 0)),
            pl.BlockSpec((x.shape[1], 2 * H), lambda i: (0, 0)),
            pl.BlockSpec((1, 2 * H), lambda i: (0, 0)),
        ],
        out_specs=[
            pl.BlockSpec((R, H), lambda i: (i, 0)),
            pl.BlockSpec((2, R, HH), lambda i: (0, i, 0)),
        ],
        out_shape=[
            jax.ShapeDtypeStruct((N, H), jnp.float32),
            jax.ShapeDtypeStruct((2, N, HH), jnp.float32),
        ],
    )(x, w, b)


def _layerB(y0p, gsp, w, b):
    return pl.pallas_call(
        _layerB_body,
        grid=(GRID,),
        in_specs=[
            pl.BlockSpec((R, H), lambda i: (i, 0)),
            pl.BlockSpec((2, R, HH), lambda i: (0, i, 0)),
            pl.BlockSpec((H, 2 * H), lambda i: (0, 0)),
            pl.BlockSpec((1, 2 * H), lambda i: (0, 0)),
        ],
        out_specs=[
            pl.BlockSpec((R, H), lambda i: (i, 0)),
            pl.BlockSpec((2, R, HH), lambda i: (0, i, 0)),
        ],
        out_shape=[
            jax.ShapeDtypeStruct((N, H), jnp.float32),
            jax.ShapeDtypeStruct((2, N, HH), jnp.float32),
        ],
    )(y0p, gsp, w, b)


def _pool_body(y0p_ref, gsp_ref, seg_ref, fw1_ref, fb1_ref, fw2_ref, fb2_ref,
               out_ref, acc, cnt):
    i = pl.program_id(0)

    @pl.when(i == 0)
    def _():
        acc[...] = jnp.zeros_like(acc)
        cnt[...] = jnp.zeros_like(cnt)

    gs = jnp.concatenate([gsp_ref[0], gsp_ref[1]], axis=1)
    x = jnp.maximum(y0p_ref[...] + gs, 0.0)
    seg = seg_ref[...].reshape(1, R)
    sel = jnp.broadcast_to(seg, (16, R)) == lax.broadcasted_iota(
        jnp.int32, (16, R), 0)
    onehot = sel.astype(jnp.float32)
    acc[...] += jnp.dot(onehot, x, preferred_element_type=jnp.float32)
    csum = jnp.sum(onehot, axis=1, keepdims=True)
    cnt[...] += jnp.broadcast_to(csum, cnt.shape)

    @pl.when(i == GRID - 1)
    def _():
        counts = jnp.maximum(cnt[:, :1], 1.0)
        mesh_feats = acc[...] / counts
        h = jnp.dot(mesh_feats, fw1_ref[...], preferred_element_type=jnp.float32)
        h = jnp.maximum(h + fb1_ref[...], 0.0)
        o = jnp.dot(h, fw2_ref[...], preferred_element_type=jnp.float32)
        o = o + fb2_ref[...]
        out_ref[...] = o[:NB, :]


def _pool(y0p, gsp, seg3, fw1, fb1, fw2, fb2):
    return pl.pallas_call(
        _pool_body,
        grid=(GRID,),
        in_specs=[
            pl.BlockSpec((R, H), lambda i: (i, 0)),
            pl.BlockSpec((2, R, HH), lambda i: (0, i, 0)),
            pl.BlockSpec((1, 1, R), lambda i: (i, 0, 0)),
            pl.BlockSpec((H, H), lambda i: (0, 0)),
            pl.BlockSpec((1, H), lambda i: (0, 0)),
            pl.BlockSpec((H, NC), lambda i: (0, 0)),
            pl.BlockSpec((1, NC), lambda i: (0, 0)),
        ],
        out_specs=pl.BlockSpec((NB, NC), lambda i: (0, 0)),
        out_shape=jax.ShapeDtypeStruct((NB, NC), jnp.float32),
        scratch_shapes=[
            pltpu.VMEM((16, H), jnp.float32),
            pltpu.VMEM((16, 128), jnp.float32),
        ],
    )(y0p, gsp, seg3, fw1, fb1, fw2, fb2)


def kernel(verts, edges, segment_ids,
           w0_1, b0_1, w1_1, b1_1,
           w0_2, b0_2, w1_2, b1_2,
           w0_3, b0_3, w1_3, b1_3,
           fc1_w, fc1_b, fc2_w, fc2_b):
    src = jnp.concatenate([edges[:, 1], edges[:, 0]])
    dst = jnp.concatenate([edges[:, 0], edges[:, 1]])
    pad = E2P - E2
    srcp = jnp.concatenate([src, jnp.zeros((pad,), jnp.int32)])
    dstp = jnp.concatenate([dst, jnp.full((pad,), PAD_ROW, jnp.int32)])
    src2 = jnp.concatenate([srcp, srcp + N])

    w1c = jnp.concatenate([w0_1.T, w1_1.T], axis=1)
    b1c = jnp.concatenate([b0_1, b1_1]).reshape(1, 2 * H)
    w2c = jnp.concatenate([w0_2.T, w1_2.T], axis=1)
    b2c = jnp.concatenate([b0_2, b1_2]).reshape(1, 2 * H)
    w3c = jnp.concatenate([w0_3.T, w1_3.T], axis=1)
    b3c = jnp.concatenate([b0_3, b1_3]).reshape(1, 2 * H)
    seg3 = segment_ids.reshape(GRID, 1, R)

    y0, y1s = _layer1(verts, w1c, b1c)
    gs = _gs_call(y1s.reshape(2 * N, HH), src2, dstp)
    y0, y1s = _layerB(y0, gs.reshape(2, N, HH), w2c, b2c)
    gs = _gs_call(y1s.reshape(2 * N, HH), src2, dstp)
    y0, y1s = _layerB(y0, gs.reshape(2, N, HH), w3c, b3c)
    gs = _gs_call(y1s.reshape(2 * N, HH), src2, dstp)

    return _pool(y0, gs.reshape(2, N, HH), seg3,
                 fc1_w.T, fc1_b.reshape(1, H),
                 fc2_w.T, fc2_b.reshape(1, NC))


# speedup vs baseline: 3.1205x; 1.4204x over previous
import functools

import jax
import jax.numpy as jnp
from jax import lax
from jax.experimental import pallas as pl
from jax.experimental.pallas import tpu as pltpu
from jax.experimental.pallas import tpu_sc as plsc

N = 10000
E = 160000
H = 256
HH = 128
NB = 10
NC = 10
R = 1000
GRID = N // R

NTILES = 16
NCORES = 2
CH = 128
NCH = 160
E2 = 2 * E
E2P = NTILES * NCH * CH
ACC_ROWS = 10240
PAD_ROW = N + 64
G = 32
NGROUPS = NCH // G
ZROWS = 16
OUT_PER_TILE = N // NTILES


def _gs_body(y1s_hbm, src2_hbm, dst_hbm, out_hbm, acc,
             src_g, dst_g, rows0, rows1, zbuf, sem0, sem1):
    c = lax.axis_index("c")
    s = lax.axis_index("s")

    pltpu.sync_copy(src2_hbm.at[(c * NTILES + s) * NGROUPS], src_g)
    pltpu.sync_copy(dst_hbm.at[s * NGROUPS], dst_g)
    pltpu.async_copy(y1s_hbm.at[src_g.at[0]], rows0, sem0)
    pltpu.async_copy(y1s_hbm.at[src_g.at[1]], rows1, sem1)

    def _zb(i, _):
        r = i // (HH // 16)
        l = i % (HH // 16)
        zbuf[r, pl.ds(l * 16, 16)] = jnp.zeros((16,), jnp.float32)
        return _
    lax.fori_loop(0, ZROWS * (HH // 16), _zb, 0)

    rows_per_tile = ACC_ROWS // NTILES

    def _zero(i, _):
        pltpu.sync_copy(zbuf, acc.at[pl.ds(s * rows_per_tile + i * ZROWS, ZROWS)])
        return _
    lax.fori_loop(0, rows_per_tile // ZROWS, _zero, 0)

    plsc.subcore_barrier()

    def _drain_start(k, rows, sem):
        pltpu.make_async_copy(y1s_hbm.at[src_g.at[0]], rows, sem).wait()
        pltpu.sync_copy(rows, acc.at[dst_g.at[k]], add=True)

        @pl.when(k + 2 < G)
        def _():
            pltpu.async_copy(y1s_hbm.at[src_g.at[k + 2]], rows, sem)

    def _pair(i, carry):
        k0 = 2 * i
        _drain_start(k0, rows0, sem0)
        _drain_start(k0 + 1, rows1, sem1)
        return carry

    for g in range(NGROUPS):
        if g > 0:
            pltpu.sync_copy(src2_hbm.at[(c * NTILES + s) * NGROUPS + g], src_g)
            pltpu.sync_copy(dst_hbm.at[s * NGROUPS + g], dst_g)
            pltpu.async_copy(y1s_hbm.at[src_g.at[0]], rows0, sem0)
            pltpu.async_copy(y1s_hbm.at[src_g.at[1]], rows1, sem1)
        lax.fori_loop(0, G // 2, _pair, 0)

    plsc.subcore_barrier()

    @pl.when(s < NTILES - 1)
    def _():
        pltpu.sync_copy(acc.at[pl.ds(s * 640, 640)],
                        out_hbm.at[pl.ds(c * N + s * 640, 640)])

    @pl.when(s == NTILES - 1)
    def _():
        pltpu.sync_copy(acc.at[pl.ds((NTILES - 1) * 640, 400)],
                        out_hbm.at[pl.ds(c * N + (NTILES - 1) * 640, 400)])


_gs_call = functools.partial(
    pl.kernel,
    out_type=jax.ShapeDtypeStruct((NCORES * N, HH), jnp.float32),
    mesh=plsc.VectorSubcoreMesh(core_axis_name="c", subcore_axis_name="s"),
    scratch_types=[
        pltpu.VMEM_SHARED((ACC_ROWS, HH), jnp.float32),
        pltpu.VMEM((G, CH), jnp.int32),
        pltpu.VMEM((G, CH), jnp.int32),
        pltpu.VMEM((CH, HH), jnp.float32),
        pltpu.VMEM((CH, HH), jnp.float32),
        pltpu.VMEM((ZROWS, HH), jnp.float32),
        pltpu.SemaphoreType.DMA,
        pltpu.SemaphoreType.DMA,
    ],
)(_gs_body)


def _layer1_body(x_ref, w_ref, b_ref, y0_ref, y1s_ref):
    y = jnp.dot(x_ref[...], w_ref[...], preferred_element_type=jnp.float32)
    y = y + b_ref[...]
    y0_ref[...] = y[:, :H]
    y1s_ref[0] = y[:, H:H + HH]
    y1s_ref[1] = y[:, H + HH:]


def _layerB_body(y0p_ref, gsp_ref, w_ref, b_ref, y0_ref, y1s_ref):
    gs = jnp.concatenate([gsp_ref[0], gsp_ref[1]], axis=1)
    x = jnp.maximum(y0p_ref[...] + gs, 0.0)
    y = jnp.dot(x, w_ref[...], preferred_element_type=jnp.float32)
    y = y + b_ref[...]
    y0_ref[...] = y[:, :H]
    y1s_ref[0] = y[:, H:H + HH]
    y1s_ref[1] = y[:, H + HH:]


def _layer1(x, w, b):
    return pl.pallas_call(
        _layer1_body,
        grid=(GRID,),
        in_specs=[
            pl.BlockSpec((R, x.shape[1]), lambda i: (i, 0)),
            pl.BlockSpec((x.shape[1], 2 * H), lambda i: (0, 0)),
            pl.BlockSpec((1, 2 * H), lambda i: (0, 0)),
        ],
        out_specs=[
            pl.BlockSpec((R, H), lambda i: (i, 0)),
            pl.BlockSpec((2, R, HH), lambda i: (0, i, 0)),
        ],
        out_shape=[
            jax.ShapeDtypeStruct((N, H), jnp.float32),
            jax.ShapeDtypeStruct((2, N, HH), jnp.float32),
        ],
    )(x, w, b)


def _layerB(y0p, gsp, w, b):
    return pl.pallas_call(
        _layerB_body,
        grid=(GRID,),
        in_specs=[
            pl.BlockSpec((R, H), lambda i: (i, 0)),
            pl.BlockSpec((2, R, HH), lambda i: (0, i, 0)),
            pl.BlockSpec((H, 2 * H), lambda i: (0, 0)),
            pl.BlockSpec((1, 2 * H), lambda i: (0, 0)),
        ],
        out_specs=[
            pl.BlockSpec((R, H), lambda i: (i, 0)),
            pl.BlockSpec((2, R, HH), lambda i: (0, i, 0)),
        ],
        out_shape=[
            jax.ShapeDtypeStruct((N, H), jnp.float32),
            jax.ShapeDtypeStruct((2, N, HH), jnp.float32),
        ],
    )(y0p, gsp, w, b)


def _pool_body(y0p_ref, gsp_ref, seg_ref, fw1_ref, fb1_ref, fw2_ref, fb2_ref,
               out_ref, acc, cnt):
    i = pl.program_id(0)

    @pl.when(i == 0)
    def _():
        acc[...] = jnp.zeros_like(acc)
        cnt[...] = jnp.zeros_like(cnt)

    gs = jnp.concatenate([gsp_ref[0], gsp_ref[1]], axis=1)
    x = jnp.maximum(y0p_ref[...] + gs, 0.0)
    seg = seg_ref[...].reshape(1, R)
    sel = jnp.broadcast_to(seg, (16, R)) == lax.broadcasted_iota(
        jnp.int32, (16, R), 0)
    onehot = sel.astype(jnp.float32)
    acc[...] += jnp.dot(onehot, x, preferred_element_type=jnp.float32)
    csum = jnp.sum(onehot, axis=1, keepdims=True)
    cnt[...] += jnp.broadcast_to(csum, cnt.shape)

    @pl.when(i == GRID - 1)
    def _():
        counts = jnp.maximum(cnt[:, :1], 1.0)
        mesh_feats = acc[...] / counts
        h = jnp.dot(mesh_feats, fw1_ref[...], preferred_element_type=jnp.float32)
        h = jnp.maximum(h + fb1_ref[...], 0.0)
        o = jnp.dot(h, fw2_ref[...], preferred_element_type=jnp.float32)
        o = o + fb2_ref[...]
        out_ref[...] = o[:NB, :]


def _pool(y0p, gsp, seg3, fw1, fb1, fw2, fb2):
    return pl.pallas_call(
        _pool_body,
        grid=(GRID,),
        in_specs=[
            pl.BlockSpec((R, H), lambda i: (i, 0)),
            pl.BlockSpec((2, R, HH), lambda i: (0, i, 0)),
            pl.BlockSpec((1, 1, R), lambda i: (i, 0, 0)),
            pl.BlockSpec((H, H), lambda i: (0, 0)),
            pl.BlockSpec((1, H), lambda i: (0, 0)),
            pl.BlockSpec((H, NC), lambda i: (0, 0)),
            pl.BlockSpec((1, NC), lambda i: (0, 0)),
        ],
        out_specs=pl.BlockSpec((NB, NC), lambda i: (0, 0)),
        out_shape=jax.ShapeDtypeStruct((NB, NC), jnp.float32),
        scratch_shapes=[
            pltpu.VMEM((16, H), jnp.float32),
            pltpu.VMEM((16, 128), jnp.float32),
        ],
    )(y0p, gsp, seg3, fw1, fb1, fw2, fb2)


def kernel(verts, edges, segment_ids,
           w0_1, b0_1, w1_1, b1_1,
           w0_2, b0_2, w1_2, b1_2,
           w0_3, b0_3, w1_3, b1_3,
           fc1_w, fc1_b, fc2_w, fc2_b):
    src = jnp.concatenate([edges[:, 1], edges[:, 0]])
    dst = jnp.concatenate([edges[:, 0], edges[:, 1]])
    pad = E2P - E2
    srcp = jnp.concatenate([src, jnp.zeros((pad,), jnp.int32)])
    dstp = jnp.concatenate([dst, jnp.full((pad,), PAD_ROW, jnp.int32)])
    src2 = jnp.concatenate([srcp, srcp + N]).reshape(
        NCORES * NTILES * NGROUPS, G, CH)
    dstp = dstp.reshape(NTILES * NGROUPS, G, CH)

    w1c = jnp.concatenate([w0_1.T, w1_1.T], axis=1)
    b1c = jnp.concatenate([b0_1, b1_1]).reshape(1, 2 * H)
    w2c = jnp.concatenate([w0_2.T, w1_2.T], axis=1)
    b2c = jnp.concatenate([b0_2, b1_2]).reshape(1, 2 * H)
    w3c = jnp.concatenate([w0_3.T, w1_3.T], axis=1)
    b3c = jnp.concatenate([b0_3, b1_3]).reshape(1, 2 * H)
    seg3 = segment_ids.reshape(GRID, 1, R)

    y0, y1s = _layer1(verts, w1c, b1c)
    gs = _gs_call(y1s.reshape(2 * N, HH), src2, dstp)
    y0, y1s = _layerB(y0, gs.reshape(2, N, HH), w2c, b2c)
    gs = _gs_call(y1s.reshape(2 * N, HH), src2, dstp)
    y0, y1s = _layerB(y0, gs.reshape(2, N, HH), w3c, b3c)
    gs = _gs_call(y1s.reshape(2 * N, HH), src2, dstp)

    return _pool(y0, gs.reshape(2, N, HH), seg3,
                 fc1_w.T, fc1_b.reshape(1, H),
                 fc2_w.T, fc2_b.reshape(1, NC))


# speedup vs baseline: 3.1227x; 1.0007x over previous
import functools

import jax
import jax.numpy as jnp
from jax import lax
from jax.experimental import pallas as pl
from jax.experimental.pallas import tpu as pltpu
from jax.experimental.pallas import tpu_sc as plsc

N = 10000
E = 160000
H = 256
HH = 128
NB = 10
NC = 10
R = 1000
GRID = N // R

NTILES = 16
NCORES = 2
CH = 128
NCH = 160
E2 = 2 * E
E2P = NTILES * NCH * CH
ACC_ROWS = 10240
PAD_ROW = N + 64
G = 32
NGROUPS = NCH // G
ZROWS = 16
OUT_PER_TILE = N // NTILES


def _gs_body(y1s_hbm, src2_hbm, dst_hbm, out_hbm, acc,
             src_g, dst_g, rows0, rows1, zbuf, sem0, sem1, ssem0, ssem1):
    c = lax.axis_index("c")
    s = lax.axis_index("s")

    pltpu.sync_copy(src2_hbm.at[(c * NTILES + s) * NGROUPS], src_g)
    pltpu.sync_copy(dst_hbm.at[s * NGROUPS], dst_g)
    pltpu.async_copy(y1s_hbm.at[src_g.at[0]], rows0, sem0)
    pltpu.async_copy(y1s_hbm.at[src_g.at[1]], rows1, sem1)

    def _zb(i, _):
        r = i // (HH // 16)
        l = i % (HH // 16)
        zbuf[r, pl.ds(l * 16, 16)] = jnp.zeros((16,), jnp.float32)
        return _
    lax.fori_loop(0, ZROWS * (HH // 16), _zb, 0)

    rows_per_tile = ACC_ROWS // NTILES

    def _zero(i, _):
        pltpu.sync_copy(zbuf, acc.at[pl.ds(s * rows_per_tile + i * ZROWS, ZROWS)])
        return _
    lax.fori_loop(0, rows_per_tile // ZROWS, _zero, 0)

    plsc.subcore_barrier()

    def _drain_start(k, rows, gsem, ssem):
        pltpu.make_async_copy(y1s_hbm.at[src_g.at[0]], rows, gsem).wait()
        pltpu.async_copy(rows, acc.at[dst_g.at[k]], ssem, add=True)

        @pl.when(k + 2 < G)
        def _():
            pltpu.make_async_copy(rows, acc.at[dst_g.at[0]], ssem).wait()
            pltpu.async_copy(y1s_hbm.at[src_g.at[k + 2]], rows, gsem)

    def _pair(i, carry):
        k0 = 2 * i
        _drain_start(k0, rows0, sem0, ssem0)
        _drain_start(k0 + 1, rows1, sem1, ssem1)
        return carry

    for g in range(NGROUPS):
        if g > 0:
            pltpu.sync_copy(src2_hbm.at[(c * NTILES + s) * NGROUPS + g], src_g)
            pltpu.sync_copy(dst_hbm.at[s * NGROUPS + g], dst_g)
            pltpu.async_copy(y1s_hbm.at[src_g.at[0]], rows0, sem0)
            pltpu.async_copy(y1s_hbm.at[src_g.at[1]], rows1, sem1)
        lax.fori_loop(0, G // 2, _pair, 0)
        pltpu.make_async_copy(rows0, acc.at[dst_g.at[0]], ssem0).wait()
        pltpu.make_async_copy(rows1, acc.at[dst_g.at[1]], ssem1).wait()

    plsc.subcore_barrier()

    @pl.when(s < NTILES - 1)
    def _():
        pltpu.sync_copy(acc.at[pl.ds(s * 640, 640)],
                        out_hbm.at[pl.ds(c * N + s * 640, 640)])

    @pl.when(s == NTILES - 1)
    def _():
        pltpu.sync_copy(acc.at[pl.ds((NTILES - 1) * 640, 400)],
                        out_hbm.at[pl.ds(c * N + (NTILES - 1) * 640, 400)])


_gs_call = functools.partial(
    pl.kernel,
    out_type=jax.ShapeDtypeStruct((NCORES * N, HH), jnp.float32),
    mesh=plsc.VectorSubcoreMesh(core_axis_name="c", subcore_axis_name="s"),
    scratch_types=[
        pltpu.VMEM_SHARED((ACC_ROWS, HH), jnp.float32),
        pltpu.VMEM((G, CH), jnp.int32),
        pltpu.VMEM((G, CH), jnp.int32),
        pltpu.VMEM((CH, HH), jnp.float32),
        pltpu.VMEM((CH, HH), jnp.float32),
        pltpu.VMEM((ZROWS, HH), jnp.float32),
        pltpu.SemaphoreType.DMA,
        pltpu.SemaphoreType.DMA,
        pltpu.SemaphoreType.DMA,
        pltpu.SemaphoreType.DMA,
    ],
)(_gs_body)


def _layer1_body(x_ref, w_ref, b_ref, y0_ref, y1s_ref):
    y = jnp.dot(x_ref[...], w_ref[...], preferred_element_type=jnp.float32)
    y = y + b_ref[...]
    y0_ref[...] = y[:, :H]
    y1s_ref[0] = y[:, H:H + HH]
    y1s_ref[1] = y[:, H + HH:]


def _layerB_body(y0p_ref, gsp_ref, w_ref, b_ref, y0_ref, y1s_ref):
    gs = jnp.concatenate([gsp_ref[0], gsp_ref[1]], axis=1)
    x = jnp.maximum(y0p_ref[...] + gs, 0.0)
    y = jnp.dot(x, w_ref[...], preferred_element_type=jnp.float32)
    y = y + b_ref[...]
    y0_ref[...] = y[:, :H]
    y1s_ref[0] = y[:, H:H + HH]
    y1s_ref[1] = y[:, H + HH:]


def _layer1(x, w, b):
    return pl.pallas_call(
        _layer1_body,
        grid=(GRID,),
        in_specs=[
            pl.BlockSpec((R, x.shape[1]), lambda i: (i, 0)),
            pl.BlockSpec((x.shape[1], 2 * H), lambda i: (0, 0)),
            pl.BlockSpec((1, 2 * H), lambda i: (0, 0)),
        ],
        out_specs=[
            pl.BlockSpec((R, H), lambda i: (i, 0)),
            pl.BlockSpec((2, R, HH), lambda i: (0, i, 0)),
        ],
        out_shape=[
            jax.ShapeDtypeStruct((N, H), jnp.float32),
            jax.ShapeDtypeStruct((2, N, HH), jnp.float32),
        ],
    )(x, w, b)


def _layerB(y0p, gsp, w, b):
    return pl.pallas_call(
        _layerB_body,
        grid=(GRID,),
        in_specs=[
            pl.BlockSpec((R, H), lambda i: (i, 0)),
            pl.BlockSpec((2, R, HH), lambda i: (0, i, 0)),
            pl.BlockSpec((H, 2 * H), lambda i: (0, 0)),
            pl.BlockSpec((1, 2 * H), lambda i: (0, 0)),
        ],
        out_specs=[
            pl.BlockSpec((R, H), lambda i: (i, 0)),
            pl.BlockSpec((2, R, HH), lambda i: (0, i, 0)),
        ],
        out_shape=[
            jax.ShapeDtypeStruct((N, H), jnp.float32),
            jax.ShapeDtypeStruct((2, N, HH), jnp.float32),
        ],
    )(y0p, gsp, w, b)


def _pool_body(y0p_ref, gsp_ref, seg_ref, fw1_ref, fb1_ref, fw2_ref, fb2_ref,
               out_ref, acc, cnt):
    i = pl.program_id(0)

    @pl.when(i == 0)
    def _():
        acc[...] = jnp.zeros_like(acc)
        cnt[...] = jnp.zeros_like(cnt)

    gs = jnp.concatenate([gsp_ref[0], gsp_ref[1]], axis=1)
    x = jnp.maximum(y0p_ref[...] + gs, 0.0)
    seg = seg_ref[...].reshape(1, R)
    sel = jnp.broadcast_to(seg, (16, R)) == lax.broadcasted_iota(
        jnp.int32, (16, R), 0)
    onehot = sel.astype(jnp.float32)
    acc[...] += jnp.dot(onehot, x, preferred_element_type=jnp.float32)
    csum = jnp.sum(onehot, axis=1, keepdims=True)
    cnt[...] += jnp.broadcast_to(csum, cnt.shape)

    @pl.when(i == GRID - 1)
    def _():
        counts = jnp.maximum(cnt[:, :1], 1.0)
        mesh_feats = acc[...] / counts
        h = jnp.dot(mesh_feats, fw1_ref[...], preferred_element_type=jnp.float32)
        h = jnp.maximum(h + fb1_ref[...], 0.0)
        o = jnp.dot(h, fw2_ref[...], preferred_element_type=jnp.float32)
        o = o + fb2_ref[...]
        out_ref[...] = o[:NB, :]


def _pool(y0p, gsp, seg3, fw1, fb1, fw2, fb2):
    return pl.pallas_call(
        _pool_body,
        grid=(GRID,),
        in_specs=[
            pl.BlockSpec((R, H), lambda i: (i, 0)),
            pl.BlockSpec((2, R, HH), lambda i: (0, i, 0)),
            pl.BlockSpec((1, 1, R), lambda i: (i, 0, 0)),
            pl.BlockSpec((H, H), lambda i: (0, 0)),
            pl.BlockSpec((1, H), lambda i: (0, 0)),
            pl.BlockSpec((H, NC), lambda i: (0, 0)),
            pl.BlockSpec((1, NC), lambda i: (0, 0)),
        ],
        out_specs=pl.BlockSpec((NB, NC), lambda i: (0, 0)),
        out_shape=jax.ShapeDtypeStruct((NB, NC), jnp.float32),
        scratch_shapes=[
            pltpu.VMEM((16, H), jnp.float32),
            pltpu.VMEM((16, 128), jnp.float32),
        ],
    )(y0p, gsp, seg3, fw1, fb1, fw2, fb2)


def kernel(verts, edges, segment_ids,
           w0_1, b0_1, w1_1, b1_1,
           w0_2, b0_2, w1_2, b1_2,
           w0_3, b0_3, w1_3, b1_3,
           fc1_w, fc1_b, fc2_w, fc2_b):
    src = jnp.concatenate([edges[:, 1], edges[:, 0]])
    dst = jnp.concatenate([edges[:, 0], edges[:, 1]])
    pad = E2P - E2
    srcp = jnp.concatenate([src, jnp.zeros((pad,), jnp.int32)])
    dstp = jnp.concatenate([dst, jnp.full((pad,), PAD_ROW, jnp.int32)])
    src2 = jnp.concatenate([srcp, srcp + N]).reshape(
        NCORES * NTILES * NGROUPS, G, CH)
    dstp = dstp.reshape(NTILES * NGROUPS, G, CH)

    w1c = jnp.concatenate([w0_1.T, w1_1.T], axis=1)
    b1c = jnp.concatenate([b0_1, b1_1]).reshape(1, 2 * H)
    w2c = jnp.concatenate([w0_2.T, w1_2.T], axis=1)
    b2c = jnp.concatenate([b0_2, b1_2]).reshape(1, 2 * H)
    w3c = jnp.concatenate([w0_3.T, w1_3.T], axis=1)
    b3c = jnp.concatenate([b0_3, b1_3]).reshape(1, 2 * H)
    seg3 = segment_ids.reshape(GRID, 1, R)

    y0, y1s = _layer1(verts, w1c, b1c)
    gs = _gs_call(y1s.reshape(2 * N, HH), src2, dstp)
    y0, y1s = _layerB(y0, gs.reshape(2, N, HH), w2c, b2c)
    gs = _gs_call(y1s.reshape(2 * N, HH), src2, dstp)
    y0, y1s = _layerB(y0, gs.reshape(2, N, HH), w3c, b3c)
    gs = _gs_call(y1s.reshape(2 * N, HH), src2, dstp)

    return _pool(y0, gs.reshape(2, N, HH), seg3,
                 fc1_w.T, fc1_b.reshape(1, H),
                 fc2_w.T, fc2_b.reshape(1, NC))
